# baseline (speedup 1.0000x reference)
import functools

import numpy as np
import jax
import jax.numpy as jnp
from jax import lax
from jax.experimental import pallas as pl
from jax.experimental.pallas import tpu as pltpu
from jax.experimental.pallas import tpu_sc as plsc

N = 512
K = 8
P_TOTAL = N * (N - 1) // 2
P_PAD = 131072
_NW = 32
_PER_W = P_PAD // _NW
_CHUNK = 128
_ROWS_PER_W = _PER_W // _CHUNK

_iu0, _iu1 = np.triu_indices(N, 1)
_FLAT_IDX = np.zeros((P_PAD,), np.int32)
_FLAT_IDX[:P_TOTAL] = (_iu0 * N + _iu1).astype(np.int32)
_FLAT_IDX_2D = _FLAT_IDX.reshape(P_PAD // _CHUNK, _CHUNK)


def _edge_conv(x, xt, wa, wb, b1, w2, b2, acc_ref, sel_ref):
    ch = 8
    for c0 in range(0, x.shape[1], ch):
        part = None
        for dd in range(c0, min(c0 + ch, x.shape[1])):
            df = x[:, dd:dd + 1] - xt[dd:dd + 1, :]
            t = df * df
            part = t if part is None else part + t
        if c0 == 0:
            acc_ref[...] = part
        else:
            acc_ref[...] = acc_ref[...] + part
    dist = acc_ref[...]

    xa = x @ wa + b1
    rows = lax.broadcasted_iota(jnp.int32, (N, N), 0)
    for k in range(K):
        m = jnp.min(dist, axis=0, keepdims=True)
        idxk = jnp.min(jnp.where(dist == m, rows, N), axis=0, keepdims=True)
        sel = rows == idxk
        dist = jnp.where(sel, jnp.float32(jnp.inf), dist)
        sel_ref[:, k * N:(k + 1) * N] = sel.astype(jnp.float32)
    xj = lax.dot_general(sel_ref[...], x, (((0,), (0,)), ((), ())))
    xr = jnp.concatenate([x] * K, axis=0)
    xar = jnp.concatenate([xa] * K, axis=0)
    h = jax.nn.relu(xar + (xj - xr) @ wb)
    h = jax.nn.relu(h @ w2 + b2)
    return jnp.max(h.reshape(K, N, h.shape[1]), axis=0)


def _transpose(x):
    rows = lax.broadcasted_iota(jnp.int32, (N, N), 0)
    colsq = lax.broadcasted_iota(jnp.int32, (N, N), 1)
    eye = (rows == colsq).astype(jnp.float32)
    return lax.dot_general(x, eye, (((0,), (0,)), ((), ())))


def _tc_body(pos_ref, post_ref,
             c1a, c1b, c1b1, c1w2, c1b2,
             c2a, c2b, c2b1, c2w2, c2b2,
             c3a, c3b, c3b1, c3w2, c3b2,
             sw1a, sw1b, sw1c, sb1, sw2, sb2,
             ew1i, ew1jt, ew1g, eb1, ew2, eb2,
             logits_ref, acc_ref, sel_ref):
    x1 = _edge_conv(pos_ref[...], post_ref[...], c1a[...], c1b[...],
                    c1b1[...], c1w2[...], c1b2[...], acc_ref, sel_ref)
    x2 = _edge_conv(x1, _transpose(x1), c2a[...], c2b[...],
                    c2b1[...], c2w2[...], c2b2[...], acc_ref, sel_ref)
    x3 = _edge_conv(x2, _transpose(x2), c3a[...], c3b[...],
                    c3b1[...], c3w2[...], c3b2[...], acc_ref, sel_ref)

    pre = x1 @ sw1a[...] + x2 @ sw1b[...] + x3 @ sw1c[...] + sb1[...]
    h = jax.nn.relu(pre)
    sf = jax.nn.relu(h @ sw2[...] + sb2[...])
    ge = jnp.max(sf, axis=0, keepdims=True)

    a = sf @ ew1i[...] + (ge @ ew1g[...] + eb1[...])
    bt = lax.dot_general(ew1jt[...], sf, (((1,), (1,)), ((), ())))
    w2c = ew2[...]
    bias2 = eb2[0, 0]

    bi = 8
    for ib in range(N // bi):
        r0 = ib * bi
        j0 = (r0 // 128) * 128
        a8 = a[r0:r0 + bi][:, :, None]
        hh = jax.nn.relu(bt[None, :, j0:] + a8)
        m = jnp.sum(hh * w2c[None, :, :], axis=1) + bias2
        logits_ref[r0:r0 + bi, j0:] = m


_TC_CALL = pl.pallas_call(
    _tc_body,
    out_shape=[jax.ShapeDtypeStruct((N, N), jnp.float32)],
    scratch_shapes=[pltpu.VMEM((N, N), jnp.float32),
                    pltpu.VMEM((N, K * N), jnp.float32)],
)


def _sc_body(idx_hbm, lgt_hbm, out_l, out_p, idx_v, lg_v, pb_v, sem):
    wid = lax.axis_index("s") * 2 + lax.axis_index("c")
    base = wid * _PER_W
    pltpu.sync_copy(idx_hbm.at[pl.ds(base, _PER_W)], idx_v)
    pltpu.async_copy(lgt_hbm.at[idx_v], lg_v, sem).wait()

    def sig(i, _):
        x = lg_v[pl.ds(i * 16, 16)]
        pb_v[pl.ds(i * 16, 16)] = 1.0 / (1.0 + jnp.exp(-x))
        return _

    lax.fori_loop(0, _PER_W // 16, sig, 0)
    pltpu.sync_copy(lg_v, out_l.at[pl.ds(base, _PER_W)])
    pltpu.sync_copy(pb_v, out_p.at[pl.ds(base, _PER_W)])


@functools.cache
def _sc_call():
    return pl.kernel(
        _sc_body,
        mesh=plsc.VectorSubcoreMesh(core_axis_name="c", subcore_axis_name="s"),
        out_type=[jax.ShapeDtypeStruct((P_PAD,), jnp.float32),
                  jax.ShapeDtypeStruct((P_PAD,), jnp.float32)],
        scratch_types=[pltpu.VMEM((_PER_W,), jnp.int32),
                       pltpu.VMEM((_PER_W,), jnp.float32),
                       pltpu.VMEM((_PER_W,), jnp.float32),
                       pltpu.SemaphoreType.DMA],
    )


def _tc_args(pos, p):
    def split(w, d):
        return w[:d], w[d:]

    c1a, c1b = split(p['c1_w1'], 3)
    c2a, c2b = split(p['c2_w1'], 32)
    c3a, c3b = split(p['c3_w1'], 128)
    return [
        pos, pos.T,
        c1a, c1b, p['c1_b1'][None], p['c1_w2'], p['c1_b2'][None],
        c2a, c2b, p['c2_b1'][None], p['c2_w2'], p['c2_b2'][None],
        c3a, c3b, p['c3_b1'][None], p['c3_w2'], p['c3_b2'][None],
        p['s_w1'][:32], p['s_w1'][32:160], p['s_w1'][160:],
        p['s_b1'][None], p['s_w2'], p['s_b2'][None],
        p['e_w1'][:128], p['e_w1'][128:256].T, p['e_w1'][256:],
        p['e_b1'][None], p['e_w2'], p['e_b2'][None],
    ]


def kernel(pos, batch, params):
    logits_m, = _TC_CALL(*_tc_args(pos, params))
    idx = jnp.asarray(_FLAT_IDX)
    lg, pb = _sc_call()(idx, logits_m.reshape(-1))
    logits = lg[:P_TOTAL].reshape(1, P_TOTAL)
    prob = pb[:P_TOTAL].reshape(1, P_TOTAL)
    return prob, logits


# speedup vs baseline: 1.1979x; 1.1979x over previous
import functools

import numpy as np
import jax
import jax.numpy as jnp
from jax import lax
from jax.experimental import pallas as pl
from jax.experimental.pallas import tpu as pltpu
from jax.experimental.pallas import tpu_sc as plsc

N = 512
K = 8
P_TOTAL = N * (N - 1) // 2
P_PAD = 131072
_NW = 32
_PER_W = P_PAD // _NW
_CHUNK = 128
_ROWS_PER_W = _PER_W // _CHUNK

_iu0, _iu1 = np.triu_indices(N, 1)
_FLAT_IDX = np.zeros((P_PAD,), np.int32)
_FLAT_IDX[:P_TOTAL] = (_iu0 * N + _iu1).astype(np.int32)
_FLAT_IDX_2D = _FLAT_IDX.reshape(P_PAD // _CHUNK, _CHUNK)


def _edge_conv(x, xt, wa, wb, b1, w2, b2, acc_ref, sel_ref):
    ch = 8
    for c0 in range(0, x.shape[1], ch):
        part = None
        for dd in range(c0, min(c0 + ch, x.shape[1])):
            df = x[:, dd:dd + 1] - xt[dd:dd + 1, :]
            t = df * df
            part = t if part is None else part + t
        if c0 == 0:
            acc_ref[...] = part
        else:
            acc_ref[...] = acc_ref[...] + part
    dist = acc_ref[...]

    xa = x @ wa + b1
    cols = lax.broadcasted_iota(jnp.int32, (N, N), 1)
    for k in range(K):
        m = jnp.min(dist, axis=1, keepdims=True)
        idxk = jnp.min(jnp.where(dist == m, cols, N), axis=1, keepdims=True)
        sel = cols == idxk
        dist = jnp.where(sel, jnp.float32(jnp.inf), dist)
        sel_ref[k * N:(k + 1) * N, :] = sel.astype(jnp.float32)
    xj = sel_ref[...] @ x
    xr = jnp.concatenate([x] * K, axis=0)
    xar = jnp.concatenate([xa] * K, axis=0)
    h = jax.nn.relu(xar + (xj - xr) @ wb)
    h = jax.nn.relu(h @ w2 + b2)
    return jnp.max(h.reshape(K, N, h.shape[1]), axis=0)


def _transpose(x):
    rows = lax.broadcasted_iota(jnp.int32, (N, N), 0)
    colsq = lax.broadcasted_iota(jnp.int32, (N, N), 1)
    eye = (rows == colsq).astype(jnp.float32)
    return lax.dot_general(x, eye, (((0,), (0,)), ((), ())))


def _tc_body(pos_ref, post_ref,
             c1a, c1b, c1b1, c1w2, c1b2,
             c2a, c2b, c2b1, c2w2, c2b2,
             c3a, c3b, c3b1, c3w2, c3b2,
             sw1a, sw1b, sw1c, sb1, sw2, sb2,
             ew1i, ew1jt, ew1g, eb1, ew2, eb2,
             logits_ref, acc_ref, sel_ref):
    x1 = _edge_conv(pos_ref[...], post_ref[...], c1a[...], c1b[...],
                    c1b1[...], c1w2[...], c1b2[...], acc_ref, sel_ref)
    x2 = _edge_conv(x1, _transpose(x1), c2a[...], c2b[...],
                    c2b1[...], c2w2[...], c2b2[...], acc_ref, sel_ref)
    x3 = _edge_conv(x2, _transpose(x2), c3a[...], c3b[...],
                    c3b1[...], c3w2[...], c3b2[...], acc_ref, sel_ref)

    pre = x1 @ sw1a[...] + x2 @ sw1b[...] + x3 @ sw1c[...] + sb1[...]
    h = jax.nn.relu(pre)
    sf = jax.nn.relu(h @ sw2[...] + sb2[...])
    ge = jnp.max(sf, axis=0, keepdims=True)

    a = sf @ ew1i[...] + (ge @ ew1g[...] + eb1[...])
    bt = lax.dot_general(ew1jt[...], sf, (((1,), (1,)), ((), ())))
    w2c = ew2[...]
    bias2 = eb2[0, 0]

    bi = 8
    for ib in range(N // bi):
        r0 = ib * bi
        j0 = (r0 // 128) * 128
        a8 = a[r0:r0 + bi][:, :, None]
        hh = jax.nn.relu(bt[None, :, j0:] + a8)
        m = jnp.sum(hh * w2c[None, :, :], axis=1) + bias2
        logits_ref[r0:r0 + bi, j0:] = m


_TC_CALL = pl.pallas_call(
    _tc_body,
    out_shape=[jax.ShapeDtypeStruct((N, N), jnp.float32)],
    scratch_shapes=[pltpu.VMEM((N, N), jnp.float32),
                    pltpu.VMEM((K * N, N), jnp.float32)],
)


def _sc_body(idx_hbm, lgt_hbm, out_l, out_p, idx_v, lg_v, pb_v, sem):
    wid = lax.axis_index("s") * 2 + lax.axis_index("c")
    base = wid * _PER_W
    pltpu.sync_copy(idx_hbm.at[pl.ds(base, _PER_W)], idx_v)
    pltpu.async_copy(lgt_hbm.at[idx_v], lg_v, sem).wait()

    def sig(i, _):
        x = lg_v[pl.ds(i * 16, 16)]
        pb_v[pl.ds(i * 16, 16)] = 1.0 / (1.0 + jnp.exp(-x))
        return _

    lax.fori_loop(0, _PER_W // 16, sig, 0)
    pltpu.sync_copy(lg_v, out_l.at[pl.ds(base, _PER_W)])
    pltpu.sync_copy(pb_v, out_p.at[pl.ds(base, _PER_W)])


@functools.cache
def _sc_call():
    return pl.kernel(
        _sc_body,
        mesh=plsc.VectorSubcoreMesh(core_axis_name="c", subcore_axis_name="s"),
        out_type=[jax.ShapeDtypeStruct((P_PAD,), jnp.float32),
                  jax.ShapeDtypeStruct((P_PAD,), jnp.float32)],
        scratch_types=[pltpu.VMEM((_PER_W,), jnp.int32),
                       pltpu.VMEM((_PER_W,), jnp.float32),
                       pltpu.VMEM((_PER_W,), jnp.float32),
                       pltpu.SemaphoreType.DMA],
    )


def _tc_args(pos, p):
    def split(w, d):
        return w[:d], w[d:]

    c1a, c1b = split(p['c1_w1'], 3)
    c2a, c2b = split(p['c2_w1'], 32)
    c3a, c3b = split(p['c3_w1'], 128)
    return [
        pos, pos.T,
        c1a, c1b, p['c1_b1'][None], p['c1_w2'], p['c1_b2'][None],
        c2a, c2b, p['c2_b1'][None], p['c2_w2'], p['c2_b2'][None],
        c3a, c3b, p['c3_b1'][None], p['c3_w2'], p['c3_b2'][None],
        p['s_w1'][:32], p['s_w1'][32:160], p['s_w1'][160:],
        p['s_b1'][None], p['s_w2'], p['s_b2'][None],
        p['e_w1'][:128], p['e_w1'][128:256].T, p['e_w1'][256:],
        p['e_b1'][None], p['e_w2'], p['e_b2'][None],
    ]


def kernel(pos, batch, params):
    logits_m, = _TC_CALL(*_tc_args(pos, params))
    idx = jnp.asarray(_FLAT_IDX)
    lg, pb = _sc_call()(idx, logits_m.reshape(-1))
    logits = lg[:P_TOTAL].reshape(1, P_TOTAL)
    prob = pb[:P_TOTAL].reshape(1, P_TOTAL)
    return prob, logits
